# Initial kernel scaffold; baseline (speedup 1.0000x reference)
import jax
import jax.numpy as jnp
from jax.experimental import pallas as pl


def kernel(sample, entity_embedding, relation_embedding):
    raise NotImplementedError("write your pallas kernel here")


# speedup vs baseline: 1.7885x; 1.7885x over previous
import functools

import jax
import jax.numpy as jnp
from jax import lax
from jax.experimental import pallas as pl
from jax.experimental.pallas import tpu as pltpu
from jax.experimental.pallas import tpu_sc as plsc

_GAMMA = 12.0
_B = 16384
_D = 128
_NC = 2
_NS = 16
_NW = _NC * _NS
_BPW = _B // _NW
_CH = 128
_NCHUNK = _BPW // _CH
_L = 16


def _build_sc_kernel():
    mesh = plsc.VectorSubcoreMesh(core_axis_name="c", subcore_axis_name="s")

    @functools.partial(
        pl.kernel,
        mesh=mesh,
        out_type=jax.ShapeDtypeStruct((_B,), jnp.float32),
        scratch_types=[
            pltpu.VMEM((3 * _NCHUNK, _CH), jnp.int32),
            pltpu.VMEM((_CH, _D), jnp.float32),
            pltpu.VMEM((_CH, _D), jnp.float32),
            pltpu.VMEM((_CH, _D), jnp.float32),
            pltpu.VMEM((_BPW,), jnp.float32),
            pltpu.SemaphoreType.DMA,
        ],
    )
    def kern(idx_hbm, ent_hbm, rel_hbm, out_hbm, idx_v, hbuf, rbuf, tbuf, obuf, sem):
        wid = lax.axis_index("s") * _NC + lax.axis_index("c")
        pltpu.sync_copy(idx_hbm.at[wid], idx_v)
        lane = lax.iota(jnp.int32, _L)
        for j in range(_NCHUNK):
            pltpu.async_copy(ent_hbm.at[idx_v.at[j]], hbuf, sem).wait()
            pltpu.async_copy(rel_hbm.at[idx_v.at[_NCHUNK + j]], rbuf, sem).wait()
            pltpu.async_copy(ent_hbm.at[idx_v.at[2 * _NCHUNK + j]], tbuf, sem).wait()
            for g in range(_CH // _L):

                def body(i, sv):
                    row = g * _L + i
                    acc = jnp.zeros((_L,), jnp.float32)
                    for k in range(_D // _L):
                        hv = hbuf[row, pl.ds(k * _L, _L)]
                        rv = rbuf[row, pl.ds(k * _L, _L)]
                        tv = tbuf[row, pl.ds(k * _L, _L)]
                        acc = acc + jnp.abs(hv + rv - tv)
                    for sh in (8, 4, 2, 1):
                        acc = acc + jnp.take(acc, lane ^ sh)
                    return jnp.where(lane == i, _GAMMA - acc, sv)

                sv = lax.fori_loop(0, _L, body, jnp.zeros((_L,), jnp.float32))
                obuf[pl.ds(j * _CH + g * _L, _L)] = sv
        pltpu.sync_copy(obuf, out_hbm.at[pl.ds(wid * _BPW, _BPW)])

    return kern


def kernel(sample, entity_embedding, relation_embedding):
    idx = sample.astype(jnp.int32).T
    idx = (
        idx.reshape(3, _NW, _NCHUNK, _CH)
        .transpose(1, 0, 2, 3)
        .reshape(_NW, 3 * _NCHUNK, _CH)
    )
    scores = _build_sc_kernel()(idx, entity_embedding, relation_embedding)
    return scores.reshape(_B, 1)
